# baseline (speedup 1.0000x reference)
import functools

import jax
import jax.numpy as jnp
from jax.experimental import pallas as pl

_NPOS = 64
_NNEG = 192
_NKEEP = _NPOS + _NNEG
_BG = 80


def _sel_loop(key, niter, col_offset, keep, gath, srcs, lane, col):
    npl = key.shape[1]

    def body(j, carry):
        key, keep, gath = carry
        m = jnp.max(key, axis=1, keepdims=True)
        sel = jnp.min(jnp.where(key == m, lane, npl), axis=1, keepdims=True)
        eq = lane == sel
        jj = j + col_offset
        at = col == jj
        keep = jnp.where(at, sel, keep)
        gath = tuple(
            jnp.where(at, jnp.sum(jnp.where(eq, s, 0.0), axis=1, keepdims=True), g)
            for g, s in zip(gath, srcs))
        key = jnp.where(eq, -3.0, key)
        return key, keep, gath

    key, keep, gath = jax.lax.fori_loop(0, niter, body, (key, keep, gath))
    return keep, gath


def _proposal_target_kernel(nreal, rois_ref, gt_ref, gtid_ref,
                            srois_ref, oh_ref, df_ref, cls_ref):
    B = rois_ref.shape[0]
    npl = rois_ref.shape[2]
    G = gt_ref.shape[2]

    y1 = rois_ref[:, 0, :]
    x1 = rois_ref[:, 1, :]
    y2 = rois_ref[:, 2, :]
    x2 = rois_ref[:, 3, :]
    area = (y2 - y1) * (x2 - x1)

    best = jnp.full((B, npl), -1.0, jnp.float32)
    bestg = jnp.zeros((B, npl), jnp.float32)
    for g in range(G):
        by1 = gt_ref[:, 0:1, g]
        bx1 = gt_ref[:, 1:2, g]
        by2 = gt_ref[:, 2:3, g]
        bx2 = gt_ref[:, 3:4, g]
        barea = (by2 - by1) * (bx2 - bx1)
        iy1 = jnp.maximum(y1, by1)
        ix1 = jnp.maximum(x1, bx1)
        iy2 = jnp.minimum(y2, by2)
        ix2 = jnp.minimum(x2, bx2)
        inter = jnp.maximum(iy2 - iy1, 0.0) * jnp.maximum(ix2 - ix1, 0.0)
        iou = inter / (area + barea - inter + 1e-8)
        upd = iou > best
        best = jnp.where(upd, iou, best)
        bestg = jnp.where(upd, jnp.float32(g), bestg)

    lane = jax.lax.broadcasted_iota(jnp.int32, (1, npl), 1)
    col = jax.lax.broadcasted_iota(jnp.int32, (B, _NKEEP), 1)
    ioumax = jnp.where(lane < nreal, best, -2.0)
    posk = jnp.where(ioumax >= 0.5, ioumax, -1.0)
    negk = jnp.where((ioumax < 0.5) & (ioumax >= 0.0), 1.0 - ioumax, -1.0)

    keep = jnp.zeros((B, _NKEEP), jnp.int32)
    gath = tuple(jnp.zeros((B, _NKEEP), jnp.float32) for _ in range(6))
    srcs = (ioumax, bestg, y1, x1, y2, x2)
    keep, gath = _sel_loop(posk, _NPOS, 0, keep, gath, srcs, lane, col)
    keep, gath = _sel_loop(negk, _NNEG, _NPOS, keep, gath, srcs, lane, col)
    s_iou, s_g, sy1, sx1, sy2, sx2 = gath
    s_posf = jnp.where(s_iou >= 0.5, 1.0, 0.0)

    gid = jnp.zeros((B, _NKEEP), jnp.float32)
    gy1 = jnp.zeros((B, _NKEEP), jnp.float32)
    gx1 = jnp.zeros((B, _NKEEP), jnp.float32)
    gy2 = jnp.zeros((B, _NKEEP), jnp.float32)
    gx2 = jnp.zeros((B, _NKEEP), jnp.float32)
    for g in range(G):
        m = s_g == jnp.float32(g)
        gid = jnp.where(m, gtid_ref[:, 0:1, g], gid)
        gy1 = jnp.where(m, gt_ref[:, 0:1, g], gy1)
        gx1 = jnp.where(m, gt_ref[:, 1:2, g], gx1)
        gy2 = jnp.where(m, gt_ref[:, 2:3, g], gy2)
        gx2 = jnp.where(m, gt_ref[:, 3:4, g], gx2)

    clsf = jnp.where(s_posf > 0.0, gid, jnp.float32(_BG))
    clsi = clsf.astype(jnp.int32)
    cls_ref[...] = clsi

    srois_ref[...] = jnp.stack([sy1, sx1, sy2, sx2], axis=-1)

    rh = jnp.maximum(sy2 - sy1, 1e-6)
    rw = jnp.maximum(sx2 - sx1, 1e-6)
    ry = sy1 + 0.5 * rh
    rx = sx1 + 0.5 * rw
    gh = jnp.maximum(gy2 - gy1, 1e-6)
    gw = jnp.maximum(gx2 - gx1, 1e-6)
    gy = gy1 + 0.5 * gh
    gx = gx1 + 0.5 * gw
    d0 = (gx - rx) / rw * 10.0
    d1 = (gy - ry) / rh * 10.0
    d2 = jnp.log(gw / rw) * 5.0
    d3 = jnp.log(gh / rh) * 5.0

    i81 = jax.lax.broadcasted_iota(jnp.int32, (1, 1, _BG + 1), 2)
    oh_ref[...] = (clsi[:, :, None] == i81).astype(jnp.float32)

    l640 = jax.lax.broadcasted_iota(jnp.int32, (1, 1, _BG * 8), 2)
    c_of_l = l640 // 8
    j_of_l = l640 % 8
    fgrep = (clsi[:, :, None] == c_of_l).astype(jnp.float32) * s_posf[:, :, None]
    erep = jnp.where(
        j_of_l < 4, 1.0,
        jnp.where(j_of_l == 4, d0[:, :, None],
                  jnp.where(j_of_l == 5, d1[:, :, None],
                            jnp.where(j_of_l == 6, d2[:, :, None],
                                      d3[:, :, None]))))
    df_ref[...] = fgrep * erep


def kernel(rois, gt_class_ids, gt_boxes):
    B, N, _ = rois.shape
    G = gt_boxes.shape[1]
    npl = ((N + 127) // 128) * 128
    rois_t = jnp.transpose(rois, (0, 2, 1))
    rois_t = jnp.pad(rois_t, ((0, 0), (0, 0), (0, npl - N)))
    gt_t = jnp.transpose(gt_boxes, (0, 2, 1))
    gtid_f = gt_class_ids.astype(jnp.float32)[:, None, :]

    out_shape = (
        jax.ShapeDtypeStruct((B, _NKEEP, 4), jnp.float32),
        jax.ShapeDtypeStruct((B, _NKEEP, _BG + 1), jnp.float32),
        jax.ShapeDtypeStruct((B, _NKEEP, _BG * 8), jnp.float32),
        jax.ShapeDtypeStruct((B, _NKEEP), jnp.int32),
    )
    fn = pl.pallas_call(
        functools.partial(_proposal_target_kernel, N),
        out_shape=out_shape,
    )
    return fn(rois_t, gt_t, gtid_f)


# speedup vs baseline: 1.0348x; 1.0348x over previous
import functools

import jax
import jax.numpy as jnp
from jax import lax
from jax.experimental import pallas as pl
from jax.experimental.pallas import tpu as pltpu
from jax.experimental.pallas import tpu_sc as plsc

_NPOS = 64
_NNEG = 192
_NKEEP = _NPOS + _NNEG
_BG = 80
_ROWPAD = 16
_NC = 2
_NS = 16
_NW = _NC * _NS


def _iou_step(y1, x1, y2, x2, area, by1, bx1, by2, bx2):
    barea = (by2 - by1) * (bx2 - bx1)
    iy1 = jnp.maximum(y1, by1)
    ix1 = jnp.maximum(x1, bx1)
    iy2 = jnp.minimum(y2, by2)
    ix2 = jnp.minimum(x2, bx2)
    inter = jnp.maximum(iy2 - iy1, 0.0) * jnp.maximum(ix2 - ix1, 0.0)
    return inter / (area + barea - inter + 1e-8)


def _sel_loop(key, niter, col_offset, keep, lane, col):
    npl = key.shape[1]

    def body(j, carry):
        key, keep = carry
        m = jnp.max(key, axis=1, keepdims=True)
        sel = jnp.min(jnp.where(key == m, lane, npl), axis=1, keepdims=True)
        keep = jnp.where(col == j + col_offset, sel, keep)
        key = jnp.where(lane == sel, -3.0, key)
        return key, keep

    _, keep = jax.lax.fori_loop(0, niter, body, (key, keep))
    return keep


def _kernel_a(nreal, rois_ref, gt_ref, keep_ref):
    B = rois_ref.shape[0]
    npl = rois_ref.shape[2]
    G = gt_ref.shape[2]

    y1 = rois_ref[:, 0, :]
    x1 = rois_ref[:, 1, :]
    y2 = rois_ref[:, 2, :]
    x2 = rois_ref[:, 3, :]
    area = (y2 - y1) * (x2 - x1)

    best = jnp.full((B, npl), -1.0, jnp.float32)
    for g in range(G):
        iou = _iou_step(y1, x1, y2, x2, area,
                        gt_ref[:, 0:1, g], gt_ref[:, 1:2, g],
                        gt_ref[:, 2:3, g], gt_ref[:, 3:4, g])
        best = jnp.maximum(best, iou)

    lane = jax.lax.broadcasted_iota(jnp.int32, (1, npl), 1)
    col = jax.lax.broadcasted_iota(jnp.int32, (B, _NKEEP), 1)
    ioumax = jnp.where(lane < nreal, best, -2.0)
    posk = jnp.where(ioumax >= 0.5, ioumax, -1.0)
    negk = jnp.where((ioumax < 0.5) & (ioumax >= 0.0), 1.0 - ioumax, -1.0)

    keep = jnp.zeros((B, _NKEEP), jnp.int32)
    keep = _sel_loop(posk, _NPOS, 0, keep, lane, col)
    keep = _sel_loop(negk, _NNEG, _NPOS, keep, lane, col)
    img = jax.lax.broadcasted_iota(jnp.int32, (B, _NKEEP), 0)
    keep_ref[...] = keep + img * npl


def _sc_gather(table, idx):
    n = idx.shape[0]
    b_per_w = n // _NW
    mesh = plsc.VectorSubcoreMesh(core_axis_name="c", subcore_axis_name="s")

    @functools.partial(
        pl.kernel, mesh=mesh,
        compiler_params=pltpu.CompilerParams(use_tc_tiling_on_sc=False),
        out_type=jax.ShapeDtypeStruct((n, _ROWPAD), jnp.float32),
        scratch_types=[
            pltpu.VMEM((b_per_w,), jnp.int32),
            pltpu.VMEM((b_per_w, _ROWPAD), jnp.float32),
            pltpu.SemaphoreType.DMA,
        ],
    )
    def k(table_hbm, idx_hbm, out_hbm, idx_v, rows_v, sem):
        wid = lax.axis_index("s") * _NC + lax.axis_index("c")
        base = wid * b_per_w
        pltpu.sync_copy(idx_hbm.at[pl.ds(base, b_per_w)], idx_v)
        pltpu.async_copy(table_hbm.at[idx_v], rows_v, sem).wait()
        pltpu.sync_copy(rows_v, out_hbm.at[pl.ds(base, b_per_w)])

    return k(table, idx)


def _kernel_c(rows_ref, gt_ref, gtid_ref, srois_ref, oh_ref, df_ref, cls_ref):
    B = rows_ref.shape[1]
    G = gt_ref.shape[2]

    sy1 = rows_ref[0]
    sx1 = rows_ref[1]
    sy2 = rows_ref[2]
    sx2 = rows_ref[3]
    area = (sy2 - sy1) * (sx2 - sx1)

    best = jnp.full((B, _NKEEP), -1.0, jnp.float32)
    bestg = jnp.zeros((B, _NKEEP), jnp.float32)
    for g in range(G):
        iou = _iou_step(sy1, sx1, sy2, sx2, area,
                        gt_ref[:, 0:1, g], gt_ref[:, 1:2, g],
                        gt_ref[:, 2:3, g], gt_ref[:, 3:4, g])
        upd = iou > best
        best = jnp.where(upd, iou, best)
        bestg = jnp.where(upd, jnp.float32(g), bestg)
    s_posf = jnp.where(best >= 0.5, 1.0, 0.0)

    gid = jnp.zeros((B, _NKEEP), jnp.float32)
    gy1 = jnp.zeros((B, _NKEEP), jnp.float32)
    gx1 = jnp.zeros((B, _NKEEP), jnp.float32)
    gy2 = jnp.zeros((B, _NKEEP), jnp.float32)
    gx2 = jnp.zeros((B, _NKEEP), jnp.float32)
    for g in range(G):
        m = bestg == jnp.float32(g)
        gid = jnp.where(m, gtid_ref[:, 0:1, g], gid)
        gy1 = jnp.where(m, gt_ref[:, 0:1, g], gy1)
        gx1 = jnp.where(m, gt_ref[:, 1:2, g], gx1)
        gy2 = jnp.where(m, gt_ref[:, 2:3, g], gy2)
        gx2 = jnp.where(m, gt_ref[:, 3:4, g], gx2)

    clsf = jnp.where(s_posf > 0.0, gid, jnp.float32(_BG))
    clsi = clsf.astype(jnp.int32)
    cls_ref[...] = clsi

    srois_ref[...] = jnp.stack([sy1, sx1, sy2, sx2], axis=-1)

    rh = jnp.maximum(sy2 - sy1, 1e-6)
    rw = jnp.maximum(sx2 - sx1, 1e-6)
    ry = sy1 + 0.5 * rh
    rx = sx1 + 0.5 * rw
    gh = jnp.maximum(gy2 - gy1, 1e-6)
    gw = jnp.maximum(gx2 - gx1, 1e-6)
    gy = gy1 + 0.5 * gh
    gx = gx1 + 0.5 * gw
    d0 = (gx - rx) / rw * 10.0
    d1 = (gy - ry) / rh * 10.0
    d2 = jnp.log(gw / rw) * 5.0
    d3 = jnp.log(gh / rh) * 5.0

    i81 = jax.lax.broadcasted_iota(jnp.int32, (1, 1, _BG + 1), 2)
    oh_ref[...] = (clsi[:, :, None] == i81).astype(jnp.float32)

    l640 = jax.lax.broadcasted_iota(jnp.int32, (1, 1, _BG * 8), 2)
    c_of_l = l640 // 8
    j_of_l = l640 % 8
    fgrep = (clsi[:, :, None] == c_of_l).astype(jnp.float32) * s_posf[:, :, None]
    erep = jnp.where(
        j_of_l < 4, 1.0,
        jnp.where(j_of_l == 4, d0[:, :, None],
                  jnp.where(j_of_l == 5, d1[:, :, None],
                            jnp.where(j_of_l == 6, d2[:, :, None],
                                      d3[:, :, None]))))
    df_ref[...] = fgrep * erep


def kernel(rois, gt_class_ids, gt_boxes):
    B, N, _ = rois.shape
    G = gt_boxes.shape[1]
    npl = ((N + 127) // 128) * 128
    rois_t = jnp.transpose(rois, (0, 2, 1))
    rois_t = jnp.pad(rois_t, ((0, 0), (0, 0), (0, npl - N)))
    gt_t = jnp.transpose(gt_boxes, (0, 2, 1))
    gtid_f = gt_class_ids.astype(jnp.float32)[:, None, :]

    keep = pl.pallas_call(
        functools.partial(_kernel_a, N),
        out_shape=jax.ShapeDtypeStruct((B, _NKEEP), jnp.int32),
    )(rois_t, gt_t)

    table = jnp.pad(rois, ((0, 0), (0, npl - N), (0, _ROWPAD - 4)))
    table = table.reshape(B * npl, _ROWPAD)
    rows = _sc_gather(table, keep.reshape(B * _NKEEP))
    rows = jnp.transpose(rows, (1, 0)).reshape(_ROWPAD, B, _NKEEP)

    out_shape = (
        jax.ShapeDtypeStruct((B, _NKEEP, 4), jnp.float32),
        jax.ShapeDtypeStruct((B, _NKEEP, _BG + 1), jnp.float32),
        jax.ShapeDtypeStruct((B, _NKEEP, _BG * 8), jnp.float32),
        jax.ShapeDtypeStruct((B, _NKEEP), jnp.int32),
    )
    return pl.pallas_call(_kernel_c, out_shape=out_shape)(rows, gt_t, gtid_f)


# speedup vs baseline: 1.1453x; 1.1068x over previous
import functools

import jax
import jax.numpy as jnp
from jax import lax
from jax.experimental import pallas as pl
from jax.experimental.pallas import tpu as pltpu
from jax.experimental.pallas import tpu_sc as plsc

_NPOS = 64
_NNEG = 192
_NKEEP = _NPOS + _NNEG
_BG = 80
_ROWPAD = 16
_NC = 2
_NS = 16
_NW = _NC * _NS


def _iou_step(y1, x1, y2, x2, area, by1, bx1, by2, bx2):
    barea = (by2 - by1) * (bx2 - bx1)
    iy1 = jnp.maximum(y1, by1)
    ix1 = jnp.maximum(x1, bx1)
    iy2 = jnp.minimum(y2, by2)
    ix2 = jnp.minimum(x2, bx2)
    inter = jnp.maximum(iy2 - iy1, 0.0) * jnp.maximum(ix2 - ix1, 0.0)
    return inter / (area + barea - inter + 1e-8)


def _sel_loop(val, niter, col_offset, keep, col, kref, mref):
    B, nb, _ = kref.shape
    kref[...] = val
    mref[...] = jnp.max(val, axis=2, keepdims=True)
    sub = jax.lax.broadcasted_iota(jnp.int32, (1, nb, 1), 1)
    lane = jax.lax.broadcasted_iota(jnp.int32, (B, 128), 1)

    def body(j, keep):
        mv = mref[...]
        m = jnp.max(mv, axis=1, keepdims=True)
        b = jnp.min(jnp.where(mv == m, sub, nb), axis=1, keepdims=True)
        rows = [kref[pl.ds(i, 1), pl.ds(b[i, 0, 0], 1), :] for i in range(B)]
        row = jnp.concatenate(rows, axis=0).reshape(B, 128)
        li = jnp.min(jnp.where(row == m.reshape(B, 1), lane, 128),
                     axis=1, keepdims=True)
        rowupd = jnp.where(lane == li, -3.0, row)
        newm = jnp.max(rowupd, axis=1, keepdims=True)
        for i in range(B):
            bi = b[i, 0, 0]
            kref[pl.ds(i, 1), pl.ds(bi, 1), :] = rowupd[i:i + 1, :].reshape(1, 1, 128)
            mref[pl.ds(i, 1), pl.ds(bi, 1), :] = newm[i:i + 1, :].reshape(1, 1, 1)
        sel = b.reshape(B, 1) * 128 + li
        return jnp.where(col == j + col_offset, sel, keep)

    return jax.lax.fori_loop(0, niter, body, keep)


def _kernel_a(nreal, rois_ref, gt_ref, keep_ref, kref, mref):
    B = rois_ref.shape[0]
    npl = rois_ref.shape[2]
    G = gt_ref.shape[2]

    y1 = rois_ref[:, 0, :]
    x1 = rois_ref[:, 1, :]
    y2 = rois_ref[:, 2, :]
    x2 = rois_ref[:, 3, :]
    area = (y2 - y1) * (x2 - x1)

    best = jnp.full((B, npl), -1.0, jnp.float32)
    for g in range(G):
        iou = _iou_step(y1, x1, y2, x2, area,
                        gt_ref[:, 0:1, g], gt_ref[:, 1:2, g],
                        gt_ref[:, 2:3, g], gt_ref[:, 3:4, g])
        best = jnp.maximum(best, iou)

    lane = jax.lax.broadcasted_iota(jnp.int32, (1, npl), 1)
    col = jax.lax.broadcasted_iota(jnp.int32, (B, _NKEEP), 1)
    ioumax = jnp.where(lane < nreal, best, -2.0)
    posk = jnp.where(ioumax >= 0.5, ioumax, -1.0)
    negk = jnp.where((ioumax < 0.5) & (ioumax >= 0.0), 1.0 - ioumax, -1.0)

    nb = npl // 128
    keep = jnp.zeros((B, _NKEEP), jnp.int32)
    keep = _sel_loop(posk.reshape(B, nb, 128), _NPOS, 0, keep, col, kref, mref)
    keep = _sel_loop(negk.reshape(B, nb, 128), _NNEG, _NPOS, keep, col, kref, mref)
    img = jax.lax.broadcasted_iota(jnp.int32, (B, _NKEEP), 0)
    keep_ref[...] = keep + img * npl


def _sc_gather(table, idx):
    n = idx.shape[0]
    b_per_w = n // _NW
    mesh = plsc.VectorSubcoreMesh(core_axis_name="c", subcore_axis_name="s")

    @functools.partial(
        pl.kernel, mesh=mesh,
        compiler_params=pltpu.CompilerParams(use_tc_tiling_on_sc=False),
        out_type=jax.ShapeDtypeStruct((n, _ROWPAD), jnp.float32),
        scratch_types=[
            pltpu.VMEM((b_per_w,), jnp.int32),
            pltpu.VMEM((b_per_w, _ROWPAD), jnp.float32),
            pltpu.SemaphoreType.DMA,
        ],
    )
    def k(table_hbm, idx_hbm, out_hbm, idx_v, rows_v, sem):
        wid = lax.axis_index("s") * _NC + lax.axis_index("c")
        base = wid * b_per_w
        pltpu.sync_copy(idx_hbm.at[pl.ds(base, b_per_w)], idx_v)
        pltpu.async_copy(table_hbm.at[idx_v], rows_v, sem).wait()
        pltpu.sync_copy(rows_v, out_hbm.at[pl.ds(base, b_per_w)])

    return k(table, idx)


def _kernel_c(rows_ref, gt_ref, gtid_ref, srois_ref, oh_ref, df_ref, cls_ref):
    B = rows_ref.shape[1]
    G = gt_ref.shape[2]

    sy1 = rows_ref[0]
    sx1 = rows_ref[1]
    sy2 = rows_ref[2]
    sx2 = rows_ref[3]
    area = (sy2 - sy1) * (sx2 - sx1)

    best = jnp.full((B, _NKEEP), -1.0, jnp.float32)
    bestg = jnp.zeros((B, _NKEEP), jnp.float32)
    for g in range(G):
        iou = _iou_step(sy1, sx1, sy2, sx2, area,
                        gt_ref[:, 0:1, g], gt_ref[:, 1:2, g],
                        gt_ref[:, 2:3, g], gt_ref[:, 3:4, g])
        upd = iou > best
        best = jnp.where(upd, iou, best)
        bestg = jnp.where(upd, jnp.float32(g), bestg)
    s_posf = jnp.where(best >= 0.5, 1.0, 0.0)

    gid = jnp.zeros((B, _NKEEP), jnp.float32)
    gy1 = jnp.zeros((B, _NKEEP), jnp.float32)
    gx1 = jnp.zeros((B, _NKEEP), jnp.float32)
    gy2 = jnp.zeros((B, _NKEEP), jnp.float32)
    gx2 = jnp.zeros((B, _NKEEP), jnp.float32)
    for g in range(G):
        m = bestg == jnp.float32(g)
        gid = jnp.where(m, gtid_ref[:, 0:1, g], gid)
        gy1 = jnp.where(m, gt_ref[:, 0:1, g], gy1)
        gx1 = jnp.where(m, gt_ref[:, 1:2, g], gx1)
        gy2 = jnp.where(m, gt_ref[:, 2:3, g], gy2)
        gx2 = jnp.where(m, gt_ref[:, 3:4, g], gx2)

    clsf = jnp.where(s_posf > 0.0, gid, jnp.float32(_BG))
    clsi = clsf.astype(jnp.int32)
    cls_ref[...] = clsi

    srois_ref[...] = jnp.stack([sy1, sx1, sy2, sx2], axis=-1)

    rh = jnp.maximum(sy2 - sy1, 1e-6)
    rw = jnp.maximum(sx2 - sx1, 1e-6)
    ry = sy1 + 0.5 * rh
    rx = sx1 + 0.5 * rw
    gh = jnp.maximum(gy2 - gy1, 1e-6)
    gw = jnp.maximum(gx2 - gx1, 1e-6)
    gy = gy1 + 0.5 * gh
    gx = gx1 + 0.5 * gw
    d0 = (gx - rx) / rw * 10.0
    d1 = (gy - ry) / rh * 10.0
    d2 = jnp.log(gw / rw) * 5.0
    d3 = jnp.log(gh / rh) * 5.0

    i81 = jax.lax.broadcasted_iota(jnp.int32, (1, 1, _BG + 1), 2)
    oh_ref[...] = (clsi[:, :, None] == i81).astype(jnp.float32)

    l640 = jax.lax.broadcasted_iota(jnp.int32, (1, 1, _BG * 8), 2)
    c_of_l = l640 // 8
    j_of_l = l640 % 8
    fgrep = (clsi[:, :, None] == c_of_l).astype(jnp.float32) * s_posf[:, :, None]
    erep = jnp.where(
        j_of_l < 4, 1.0,
        jnp.where(j_of_l == 4, d0[:, :, None],
                  jnp.where(j_of_l == 5, d1[:, :, None],
                            jnp.where(j_of_l == 6, d2[:, :, None],
                                      d3[:, :, None]))))
    df_ref[...] = fgrep * erep


def kernel(rois, gt_class_ids, gt_boxes):
    B, N, _ = rois.shape
    G = gt_boxes.shape[1]
    npl = ((N + 127) // 128) * 128
    rois_t = jnp.transpose(rois, (0, 2, 1))
    rois_t = jnp.pad(rois_t, ((0, 0), (0, 0), (0, npl - N)))
    gt_t = jnp.transpose(gt_boxes, (0, 2, 1))
    gtid_f = gt_class_ids.astype(jnp.float32)[:, None, :]

    keep = pl.pallas_call(
        functools.partial(_kernel_a, N),
        out_shape=jax.ShapeDtypeStruct((B, _NKEEP), jnp.int32),
        scratch_shapes=[
            pltpu.VMEM((B, npl // 128, 128), jnp.float32),
            pltpu.VMEM((B, npl // 128, 1), jnp.float32),
        ],
    )(rois_t, gt_t)

    table = jnp.pad(rois, ((0, 0), (0, npl - N), (0, _ROWPAD - 4)))
    table = table.reshape(B * npl, _ROWPAD)
    rows = _sc_gather(table, keep.reshape(B * _NKEEP))
    rows = jnp.transpose(rows, (1, 0)).reshape(_ROWPAD, B, _NKEEP)

    out_shape = (
        jax.ShapeDtypeStruct((B, _NKEEP, 4), jnp.float32),
        jax.ShapeDtypeStruct((B, _NKEEP, _BG + 1), jnp.float32),
        jax.ShapeDtypeStruct((B, _NKEEP, _BG * 8), jnp.float32),
        jax.ShapeDtypeStruct((B, _NKEEP), jnp.int32),
    )
    return pl.pallas_call(_kernel_c, out_shape=out_shape)(rows, gt_t, gtid_f)


# speedup vs baseline: 2.2373x; 1.9535x over previous
import functools

import jax
import jax.numpy as jnp
from jax.experimental import pallas as pl
from jax.experimental.pallas import tpu as pltpu

_NPOS = 64
_NNEG = 192
_NKEEP = _NPOS + _NNEG
_BG = 80


def _iou_step(y1, x1, y2, x2, area, by1, bx1, by2, bx2):
    barea = (by2 - by1) * (bx2 - bx1)
    iy1 = jnp.maximum(y1, by1)
    ix1 = jnp.maximum(x1, bx1)
    iy2 = jnp.minimum(y2, by2)
    ix2 = jnp.minimum(x2, bx2)
    inter = jnp.maximum(iy2 - iy1, 0.0) * jnp.maximum(ix2 - ix1, 0.0)
    return inter / (area + barea - inter + 1e-8)


def _sel_loop(val, niter, col_offset, carry, col, rois_ref, kref, mref):
    B, nb, _ = kref.shape
    kref[...] = val
    mref[...] = jnp.max(val, axis=2, keepdims=True)
    sub = jax.lax.broadcasted_iota(jnp.int32, (1, nb, 1), 1)
    lane = jax.lax.broadcasted_iota(jnp.int32, (B, 128), 1)

    def body(j, carry):
        keep, sy1, sx1, sy2, sx2 = carry
        mv = mref[...]
        m = jnp.max(mv, axis=1, keepdims=True)
        b = jnp.min(jnp.where(mv == m, sub, nb), axis=1, keepdims=True)
        bs = [b[i, 0, 0] for i in range(B)]
        row = jnp.concatenate(
            [kref[pl.ds(i, 1), pl.ds(bs[i], 1), :] for i in range(B)],
            axis=0).reshape(B, 128)
        crows = []
        for c in range(4):
            crows.append(jnp.concatenate(
                [rois_ref[pl.ds(i, 1), pl.ds(c, 1), pl.ds(bs[i], 1), :]
                 for i in range(B)], axis=0).reshape(B, 128))
        li = jnp.min(jnp.where(row == m.reshape(B, 1), lane, 128),
                     axis=1, keepdims=True)
        eqsel = lane == li
        rowupd = jnp.where(eqsel, -3.0, row)
        newm = jnp.max(rowupd, axis=1, keepdims=True)
        for i in range(B):
            kref[pl.ds(i, 1), pl.ds(bs[i], 1), :] = rowupd[i:i + 1, :].reshape(1, 1, 128)
            mref[pl.ds(i, 1), pl.ds(bs[i], 1), :] = newm[i:i + 1, :].reshape(1, 1, 1)
        sel = b.reshape(B, 1) * 128 + li
        at = col == j + col_offset
        vy1, vx1, vy2, vx2 = (
            jnp.sum(jnp.where(eqsel, cr, 0.0), axis=1, keepdims=True)
            for cr in crows)
        return (jnp.where(at, sel, keep),
                jnp.where(at, vy1, sy1), jnp.where(at, vx1, sx1),
                jnp.where(at, vy2, sy2), jnp.where(at, vx2, sx2))

    return jax.lax.fori_loop(0, niter, body, carry)


def _pt_kernel(nreal, rois_ref, gt_ref, gtid_ref,
               srois_ref, oh_ref, df_ref, cls_ref, kref, mref):
    B = rois_ref.shape[0]
    nb = rois_ref.shape[2]
    npl = nb * 128
    G = gt_ref.shape[2]

    y1 = rois_ref[:, 0].reshape(B, npl)
    x1 = rois_ref[:, 1].reshape(B, npl)
    y2 = rois_ref[:, 2].reshape(B, npl)
    x2 = rois_ref[:, 3].reshape(B, npl)
    area = (y2 - y1) * (x2 - x1)

    best = jnp.full((B, npl), -1.0, jnp.float32)
    for g in range(G):
        iou = _iou_step(y1, x1, y2, x2, area,
                        gt_ref[:, 0:1, g], gt_ref[:, 1:2, g],
                        gt_ref[:, 2:3, g], gt_ref[:, 3:4, g])
        best = jnp.maximum(best, iou)

    lane = jax.lax.broadcasted_iota(jnp.int32, (1, npl), 1)
    col = jax.lax.broadcasted_iota(jnp.int32, (B, _NKEEP), 1)
    ioumax = jnp.where(lane < nreal, best, -2.0)
    posk = jnp.where(ioumax >= 0.5, ioumax, -1.0)
    negk = jnp.where((ioumax < 0.5) & (ioumax >= 0.0), 1.0 - ioumax, -1.0)

    z = jnp.zeros((B, _NKEEP), jnp.float32)
    carry = (jnp.zeros((B, _NKEEP), jnp.int32), z, z, z, z)
    carry = _sel_loop(posk.reshape(B, nb, 128), _NPOS, 0, carry, col,
                      rois_ref, kref, mref)
    carry = _sel_loop(negk.reshape(B, nb, 128), _NNEG, _NPOS, carry, col,
                      rois_ref, kref, mref)
    _, sy1, sx1, sy2, sx2 = carry

    sarea = (sy2 - sy1) * (sx2 - sx1)
    sbest = jnp.full((B, _NKEEP), -1.0, jnp.float32)
    bestg = jnp.zeros((B, _NKEEP), jnp.float32)
    for g in range(G):
        iou = _iou_step(sy1, sx1, sy2, sx2, sarea,
                        gt_ref[:, 0:1, g], gt_ref[:, 1:2, g],
                        gt_ref[:, 2:3, g], gt_ref[:, 3:4, g])
        upd = iou > sbest
        sbest = jnp.where(upd, iou, sbest)
        bestg = jnp.where(upd, jnp.float32(g), bestg)
    s_posf = jnp.where(sbest >= 0.5, 1.0, 0.0)

    gid = jnp.zeros((B, _NKEEP), jnp.float32)
    gy1 = jnp.zeros((B, _NKEEP), jnp.float32)
    gx1 = jnp.zeros((B, _NKEEP), jnp.float32)
    gy2 = jnp.zeros((B, _NKEEP), jnp.float32)
    gx2 = jnp.zeros((B, _NKEEP), jnp.float32)
    for g in range(G):
        eq = bestg == jnp.float32(g)
        gid = jnp.where(eq, gtid_ref[:, 0:1, g], gid)
        gy1 = jnp.where(eq, gt_ref[:, 0:1, g], gy1)
        gx1 = jnp.where(eq, gt_ref[:, 1:2, g], gx1)
        gy2 = jnp.where(eq, gt_ref[:, 2:3, g], gy2)
        gx2 = jnp.where(eq, gt_ref[:, 3:4, g], gx2)

    clsf = jnp.where(s_posf > 0.0, gid, jnp.float32(_BG))
    clsi = clsf.astype(jnp.int32)
    cls_ref[...] = clsi

    srois_ref[...] = jnp.stack([sy1, sx1, sy2, sx2], axis=-1)

    rh = jnp.maximum(sy2 - sy1, 1e-6)
    rw = jnp.maximum(sx2 - sx1, 1e-6)
    ry = sy1 + 0.5 * rh
    rx = sx1 + 0.5 * rw
    gh = jnp.maximum(gy2 - gy1, 1e-6)
    gw = jnp.maximum(gx2 - gx1, 1e-6)
    gy = gy1 + 0.5 * gh
    gx = gx1 + 0.5 * gw
    d0 = (gx - rx) / rw * 10.0
    d1 = (gy - ry) / rh * 10.0
    d2 = jnp.log(gw / rw) * 5.0
    d3 = jnp.log(gh / rh) * 5.0

    i81 = jax.lax.broadcasted_iota(jnp.int32, (1, 1, _BG + 1), 2)
    oh_ref[...] = (clsi[:, :, None] == i81).astype(jnp.float32)

    l640 = jax.lax.broadcasted_iota(jnp.int32, (1, 1, _BG * 8), 2)
    c_of_l = l640 // 8
    j_of_l = l640 % 8
    fgrep = (clsi[:, :, None] == c_of_l).astype(jnp.float32) * s_posf[:, :, None]
    erep = jnp.where(
        j_of_l < 4, 1.0,
        jnp.where(j_of_l == 4, d0[:, :, None],
                  jnp.where(j_of_l == 5, d1[:, :, None],
                            jnp.where(j_of_l == 6, d2[:, :, None],
                                      d3[:, :, None]))))
    df_ref[...] = fgrep * erep


def kernel(rois, gt_class_ids, gt_boxes):
    B, N, _ = rois.shape
    G = gt_boxes.shape[1]
    npl = ((N + 127) // 128) * 128
    nb = npl // 128
    rois_t = jnp.transpose(rois, (0, 2, 1))
    rois_t = jnp.pad(rois_t, ((0, 0), (0, 0), (0, npl - N)))
    rois_t = rois_t.reshape(B, 4, nb, 128)
    gt_t = jnp.transpose(gt_boxes, (0, 2, 1))
    gtid_f = gt_class_ids.astype(jnp.float32)[:, None, :]

    out_shape = (
        jax.ShapeDtypeStruct((B, _NKEEP, 4), jnp.float32),
        jax.ShapeDtypeStruct((B, _NKEEP, _BG + 1), jnp.float32),
        jax.ShapeDtypeStruct((B, _NKEEP, _BG * 8), jnp.float32),
        jax.ShapeDtypeStruct((B, _NKEEP), jnp.int32),
    )
    return pl.pallas_call(
        functools.partial(_pt_kernel, N),
        out_shape=out_shape,
        scratch_shapes=[
            pltpu.VMEM((B, nb, 128), jnp.float32),
            pltpu.VMEM((B, nb, 1), jnp.float32),
        ],
    )(rois_t, gt_t, gtid_f)
